# baseline (speedup 1.0000x reference)
import functools

import jax
import jax.numpy as jnp
from jax import lax
from jax.experimental import pallas as pl
from jax.experimental.pallas import tpu as pltpu
from jax.experimental.pallas import tpu_sc as plsc

NC = 2
NS = 16
NW = NC * NS
L = 16

CB = 128


@functools.partial(jax.jit, static_argnames=("hist", "d", "batch"))
def _sc_gather_t(ids_t, table_pairs, hist, d, batch):
    mesh = plsc.VectorSubcoreMesh(
        core_axis_name="c", subcore_axis_name="s", num_cores=NC, num_subcores=NS
    )

    @functools.partial(
        pl.kernel,
        mesh=mesh,
        out_type=jax.ShapeDtypeStruct((hist, d, batch), jnp.float32),
        scratch_types=[
            pltpu.VMEM((hist, CB), jnp.int32),
            pltpu.VMEM((2, CB), jnp.int32),
            pltpu.VMEM((CB, 2 * d), jnp.float32),
            pltpu.VMEM((CB, 2 * d), jnp.float32),
            pltpu.VMEM((d, CB), jnp.float32),
            pltpu.VMEM((d, CB), jnp.float32),
            pltpu.SemaphoreType.DMA,
            pltpu.SemaphoreType.DMA,
        ],
        compiler_params=pltpu.CompilerParams(
            needs_layout_passes=False, disable_bounds_checks=True
        ),
    )
    def k(ids_hbm, tab_hbm, out_hbm, ids_v, pair_v, buf_v0, buf_v1, outt_v0,
          outt_v1, gsem, wsem):
        bufs = (buf_v0, buf_v1)
        outts = (outt_v0, outt_v1)
        w = lax.axis_index("s") * NC + lax.axis_index("c")
        b0 = w * CB
        pltpu.sync_copy(ids_hbm.at[:, pl.ds(b0, CB)], ids_v)

        def compute_pairs(h, pb):
            for g in range(CB // L):
                raw = ids_v[h, pl.ds(g * L, L)]
                pair_v[pb, pl.ds(g * L, L)] = lax.shift_right_logical(raw, 1)

        def gather_desc(pb):
            return pltpu.make_async_copy(
                tab_hbm.at[pair_v.at[pb]], bufs[pb], gsem
            )

        def write_desc(h, pb):
            return pltpu.make_async_copy(
                outts[pb], out_hbm.at[h, :, pl.ds(b0, CB)], wsem
            )

        def transpose(h, pb):
            rows_c = [lax.iota(jnp.int32, L) + kk * L for kk in range(d // L)]
            for g in range(CB // L):
                raw = ids_v[h, pl.ds(g * L, L)]
                parvec = lax.shift_left(jnp.bitwise_and(raw, 1), 6)
                for i in range(L):
                    b = g * L + i
                    par = parvec[i]
                    cols = jnp.full((L,), b, jnp.int32)
                    for kk in range(d // L):
                        vals = bufs[pb][b, pl.ds(par + kk * L, L)]
                        plsc.store_scatter(outts[pb], [rows_c[kk], cols], vals)

        compute_pairs(0, 0)
        gather_desc(0).start()

        @pl.loop(0, hist, step=2)
        def _(h0):
            for pb in range(2):
                h = h0 + pb

                @pl.when(h < hist - 1)
                def _():
                    compute_pairs(h + 1, 1 - pb)
                    gather_desc(1 - pb).start()

                gather_desc(pb).wait()

                @pl.when(h >= 2)
                def _():
                    write_desc(h - 2, pb).wait()

                transpose(h, pb)
                write_desc(h, pb).start()

        write_desc(hist - 2, 0).wait()
        write_desc(hist - 1, 1).wait()

    return k(ids_t, table_pairs)


def kernel(input_ids, embedding):
    batch, hist = input_ids.shape
    vocab, d = embedding.shape
    assert batch == NW * CB and d == 64 and vocab % 2 == 0 and hist % 2 == 0
    ids_t = input_ids.T
    table_pairs = embedding.reshape(vocab // 2, 2 * d)
    out_t = _sc_gather_t(ids_t, table_pairs, hist, d, batch)
    return jnp.transpose(out_t, (2, 0, 1))


# speedup vs baseline: 1.3309x; 1.3309x over previous
import functools

import jax
import jax.numpy as jnp
from jax import lax
from jax.experimental import pallas as pl
from jax.experimental.pallas import tpu as pltpu
from jax.experimental.pallas import tpu_sc as plsc

NC = 2
NS = 16
NW = NC * NS

CH = 128
KG = 4
SUP = CH * KG
NBUF = 2


@functools.partial(jax.jit, static_argnames=("n_per_w", "n_ch", "d"))
def _sc_gather(idx3, table, n_per_w, n_ch, d):
    n = idx3.shape[0] * idx3.shape[1] * idx3.shape[2]
    n_sup = n_per_w // SUP

    mesh = plsc.VectorSubcoreMesh(
        core_axis_name="c", subcore_axis_name="s", num_cores=NC, num_subcores=NS
    )

    @functools.partial(
        pl.kernel,
        mesh=mesh,
        out_type=jax.ShapeDtypeStruct((n, d), jnp.float32),
        scratch_types=[
            pltpu.VMEM((n_ch, CH), jnp.int32),
            pltpu.VMEM((NBUF, SUP, d), jnp.float32),
            pltpu.SemaphoreType.DMA,
            pltpu.SemaphoreType.DMA,
        ],
        compiler_params=pltpu.CompilerParams(use_tc_tiling_on_sc=False),
    )
    def k(idx_hbm, table_hbm, out_hbm, idx_v, rows_v, gsem, wsem):
        wid = lax.axis_index("s") * NC + lax.axis_index("c")
        base = wid * n_per_w
        pltpu.sync_copy(idx_hbm.at[wid], idx_v)

        def fire_gathers(t, b):
            for g in range(KG):
                pltpu.async_copy(
                    table_hbm.at[idx_v.at[t * KG + g]],
                    rows_v.at[b].at[pl.ds(g * CH, CH)],
                    gsem,
                )

        def drain_gathers(t, b):
            for g in range(KG):
                pltpu.make_async_copy(
                    table_hbm.at[idx_v.at[t * KG + g]],
                    rows_v.at[b].at[pl.ds(g * CH, CH)],
                    gsem,
                ).wait()

        def write_desc(t, b):
            return pltpu.make_async_copy(
                rows_v.at[b], out_hbm.at[pl.ds(base + t * SUP, SUP)], wsem
            )

        @pl.loop(0, n_sup, step=NBUF)
        def _(t0):
            for b in range(NBUF):
                t = t0 + b

                @pl.when(t >= NBUF)
                def _():
                    write_desc(t - NBUF, b).wait()

                fire_gathers(t, b)
                drain_gathers(t, b)
                write_desc(t, b).start()

        for b in range(NBUF):
            write_desc(n_sup - NBUF + b, b).wait()

    return k(idx3, table)


def kernel(input_ids, embedding):
    batch, hist = input_ids.shape
    vocab, d = embedding.shape
    n = batch * hist
    assert n % (NW * SUP * NBUF) == 0
    n_per_w = n // NW
    n_ch = n_per_w // CH
    idx3 = input_ids.reshape(NW, n_ch, CH)
    out = _sc_gather(idx3, embedding, n_per_w, n_ch, d)
    return out.reshape(batch, hist, d)


# speedup vs baseline: 1.3320x; 1.0008x over previous
import functools

import jax
import jax.numpy as jnp
from jax import lax
from jax.experimental import pallas as pl
from jax.experimental.pallas import tpu as pltpu
from jax.experimental.pallas import tpu_sc as plsc

NC = 2
NS = 16
NW = NC * NS

CH = 128
KG = 4
SUP = CH * KG
NBUF = 2


@functools.partial(jax.jit, static_argnames=("n_per_w", "n_ch", "d"))
def _sc_gather(idx3, table, n_per_w, n_ch, d):
    n = idx3.shape[0] * idx3.shape[1] * idx3.shape[2]
    n_sup = n_per_w // SUP

    mesh = plsc.VectorSubcoreMesh(
        core_axis_name="c", subcore_axis_name="s", num_cores=NC, num_subcores=NS
    )

    @functools.partial(
        pl.kernel,
        mesh=mesh,
        out_type=jax.ShapeDtypeStruct((n, d), jnp.float32),
        scratch_types=[
            pltpu.VMEM((n_ch, CH), jnp.int32),
            pltpu.VMEM((NBUF, SUP, d), jnp.float32),
            pltpu.SemaphoreType.DMA,
            pltpu.SemaphoreType.DMA,
        ],
        compiler_params=pltpu.CompilerParams(
            use_tc_tiling_on_sc=False, disable_bounds_checks=True
        ),
    )
    def k(idx_hbm, table_hbm, out_hbm, idx_v, rows_v, gsem, wsem):
        wid = lax.axis_index("s") * NC + lax.axis_index("c")
        base = wid * n_per_w
        pltpu.sync_copy(idx_hbm.at[wid], idx_v)

        def fire_gathers(t, b):
            for g in range(KG):
                pltpu.async_copy(
                    table_hbm.at[idx_v.at[t * KG + g]],
                    rows_v.at[b].at[pl.ds(g * CH, CH)],
                    gsem,
                )

        def drain_gathers(t, b):
            for g in range(KG):
                pltpu.make_async_copy(
                    table_hbm.at[idx_v.at[t * KG + g]],
                    rows_v.at[b].at[pl.ds(g * CH, CH)],
                    gsem,
                ).wait()

        def write_desc(t, b):
            return pltpu.make_async_copy(
                rows_v.at[b], out_hbm.at[pl.ds(base + t * SUP, SUP)], wsem
            )

        @pl.loop(0, n_sup, step=NBUF)
        def _(t0):
            for b in range(NBUF):
                t = t0 + b

                @pl.when(t >= NBUF)
                def _():
                    write_desc(t - NBUF, b).wait()

                fire_gathers(t, b)
                drain_gathers(t, b)
                write_desc(t, b).start()

        for b in range(NBUF):
            write_desc(n_sup - NBUF + b, b).wait()

    return k(idx3, table)


def kernel(input_ids, embedding):
    batch, hist = input_ids.shape
    vocab, d = embedding.shape
    n = batch * hist
    assert n % (NW * SUP * NBUF) == 0
    n_per_w = n // NW
    n_ch = n_per_w // CH
    idx3 = input_ids.reshape(NW, n_ch, CH)
    out = _sc_gather(idx3, embedding, n_per_w, n_ch, d)
    return out.reshape(batch, hist, d)
